# Initial kernel scaffold; baseline (speedup 1.0000x reference)
import jax
import jax.numpy as jnp
from jax.experimental import pallas as pl


def kernel(input1, input2, embedding_table):
    raise NotImplementedError("write your pallas kernel here")


# speedup vs baseline: 17.1044x; 17.1044x over previous
import functools

import jax
import jax.numpy as jnp
from jax import lax
from jax.experimental import pallas as pl
from jax.experimental.pallas import tpu as pltpu
from jax.experimental.pallas import tpu_sc as plsc

_B = 4096
_LSEQ = 20
_EMB = 512
_DIM = 4
_NC, _NS, _LANES = 2, 16, 16
_NW = _NC * _NS
_ROWS_PER_W = _B // _NW
_GROUPS = _ROWS_PER_W // _LANES


def _splat(v):
    return jnp.full((_LANES,), v, jnp.int32)


@functools.partial(
    pl.kernel,
    out_type=jax.ShapeDtypeStruct((_B * _DIM,), jnp.float32),
    mesh=plsc.VectorSubcoreMesh(
        core_axis_name="c", subcore_axis_name="s",
        num_cores=_NC, num_subcores=_NS),
    compiler_params=pltpu.CompilerParams(needs_layout_passes=False),
    scratch_types=[
        pltpu.VMEM((_ROWS_PER_W * _LSEQ,), jnp.int32),
        pltpu.VMEM((_ROWS_PER_W * _LSEQ,), jnp.int32),
        pltpu.VMEM((_EMB * _DIM,), jnp.float32),
        pltpu.VMEM((_ROWS_PER_W * _DIM,), jnp.float32),
    ],
)
def _edit_distance_kernel(in1_hbm, in2_hbm, table_hbm, out_hbm,
                          in1_v, in2_v, table_v, out_v):
    wid = lax.axis_index("s") * _NC + lax.axis_index("c")
    tok_base = wid * _ROWS_PER_W * _LSEQ
    out_base = wid * _ROWS_PER_W * _DIM
    pltpu.sync_copy(in1_hbm.at[pl.ds(tok_base, _ROWS_PER_W * _LSEQ)], in1_v)
    pltpu.sync_copy(in2_hbm.at[pl.ds(tok_base, _ROWS_PER_W * _LSEQ)], in2_v)
    pltpu.sync_copy(table_hbm, table_v)

    lane = lax.iota(jnp.int32, _LANES)

    def group_body(g, carry):
        row_idx = g * _LANES + lane
        tok_idx = row_idx * _LSEQ
        b = [plsc.load_gather(in2_v, [tok_idx + _splat(j)])
             for j in range(_LSEQ)]
        row = [_splat(j) for j in range(_LSEQ + 1)]
        one = _splat(1)
        for i in range(1, _LSEQ + 1):
            ai = plsc.load_gather(in1_v, [tok_idx + _splat(i - 1)])
            prev_diag = row[0]
            row[0] = _splat(i)
            for j in range(1, _LSEQ + 1):
                tmp = row[j]
                sub = jnp.where(ai == b[j - 1], prev_diag, prev_diag + one)
                row[j] = jnp.minimum(
                    jnp.minimum(row[j], row[j - 1]) + one, sub)
                prev_diag = tmp
        dist = jnp.clip(row[_LSEQ], 0, _EMB - 1)
        emb_idx = dist * _DIM
        out_idx = row_idx * _DIM
        for e in range(_DIM):
            vals = plsc.load_gather(table_v, [emb_idx + _splat(e)])
            plsc.store_scatter(out_v, [out_idx + _splat(e)], vals)
        return carry

    lax.fori_loop(0, _GROUPS, group_body, 0)
    pltpu.sync_copy(out_v, out_hbm.at[pl.ds(out_base, _ROWS_PER_W * _DIM)])


def kernel(input1, input2, embedding_table):
    out_flat = _edit_distance_kernel(
        input1.reshape(-1), input2.reshape(-1), embedding_table.reshape(-1))
    return out_flat.reshape(_B, _DIM)


# speedup vs baseline: 17.2270x; 1.0072x over previous
import functools

import jax
import jax.numpy as jnp
from jax import lax
from jax.experimental import pallas as pl
from jax.experimental.pallas import tpu as pltpu
from jax.experimental.pallas import tpu_sc as plsc

_B = 4096
_LSEQ = 20
_EMB = 512
_DIM = 4
_NC, _NS, _LANES = 2, 16, 16
_NW = _NC * _NS
_ROWS_PER_W = _B // _NW
_PACK = 2
_GROUP_ROWS = _LANES * _PACK
_GROUPS = _ROWS_PER_W // _GROUP_ROWS

_ILV = plsc.PackFormat.INTERLEAVED


def _splat16(v):
    return jnp.full((_LANES * _PACK,), v, jnp.int16)


@functools.partial(
    pl.kernel,
    out_type=jax.ShapeDtypeStruct((_B * _DIM,), jnp.float32),
    mesh=plsc.VectorSubcoreMesh(
        core_axis_name="c", subcore_axis_name="s",
        num_cores=_NC, num_subcores=_NS),
    compiler_params=pltpu.CompilerParams(needs_layout_passes=False),
    scratch_types=[
        pltpu.VMEM((_ROWS_PER_W * _LSEQ,), jnp.int32),
        pltpu.VMEM((_ROWS_PER_W * _LSEQ,), jnp.int32),
        pltpu.VMEM((_EMB * _DIM,), jnp.float32),
        pltpu.VMEM((_ROWS_PER_W * _DIM,), jnp.float32),
    ],
)
def _edit_distance_kernel(in1_hbm, in2_hbm, table_hbm, out_hbm,
                          in1_v, in2_v, table_v, out_v):
    wid = lax.axis_index("s") * _NC + lax.axis_index("c")
    tok_base = wid * _ROWS_PER_W * _LSEQ
    out_base = wid * _ROWS_PER_W * _DIM
    pltpu.sync_copy(in1_hbm.at[pl.ds(tok_base, _ROWS_PER_W * _LSEQ)], in1_v)
    pltpu.sync_copy(in2_hbm.at[pl.ds(tok_base, _ROWS_PER_W * _LSEQ)], in2_v)
    pltpu.sync_copy(table_hbm, table_v)

    lane = lax.iota(jnp.int32, _LANES)
    lane20 = lane * _LSEQ

    def tok_pack(ref, g, t):
        halves = [plsc.load_gather(
            ref, [lane20 + jnp.full((_LANES,), (g * 2 + h) * _LANES * _LSEQ + t,
                                    jnp.int32)])
            for h in range(_PACK)]
        return plsc.pack(halves[0], halves[1], format=_ILV,
                         preferred_element_type=jnp.int16)

    def group_body(g, carry):
        b = [tok_pack(in2_v, g, j) for j in range(_LSEQ)]
        row = [_splat16(j) for j in range(_LSEQ + 1)]
        one = _splat16(1)
        for i in range(1, _LSEQ + 1):
            ai = tok_pack(in1_v, g, i - 1)
            prev_diag = row[0]
            row[0] = _splat16(i)
            for j in range(1, _LSEQ + 1):
                tmp = row[j]
                cost = jnp.minimum(ai ^ b[j - 1], one)
                row[j] = jnp.minimum(
                    jnp.minimum(row[j], row[j - 1]) + one, prev_diag + cost)
                prev_diag = tmp
        d0, d1 = plsc.unpack(row[_LSEQ], format=_ILV)
        for h, d in enumerate((d0, d1)):
            dist = jnp.clip(d.astype(jnp.int32), 0, _EMB - 1)
            emb_idx = dist * _DIM
            out_idx = (lane + (g * 2 + h) * _LANES) * _DIM
            for e in range(_DIM):
                vals = plsc.load_gather(
                    table_v, [emb_idx + jnp.full((_LANES,), e, jnp.int32)])
                plsc.store_scatter(
                    out_v, [out_idx + jnp.full((_LANES,), e, jnp.int32)], vals)
        return carry

    lax.fori_loop(0, _GROUPS, group_body, 0)
    pltpu.sync_copy(out_v, out_hbm.at[pl.ds(out_base, _ROWS_PER_W * _DIM)])


def kernel(input1, input2, embedding_table):
    out_flat = _edit_distance_kernel(
        input1.reshape(-1), input2.reshape(-1), embedding_table.reshape(-1))
    return out_flat.reshape(_B, _DIM)
